# baseline (speedup 1.0000x reference)
import functools

import jax
import jax.numpy as jnp
import numpy as np
from jax import lax
from jax.experimental import pallas as pl
from jax.experimental.pallas import tpu as pltpu
from jax.experimental.pallas import tpu_sc as plsc

F32 = jnp.float32
I32 = jnp.int32

NN = 10000
NE = 320000
FEAT = 128
NH = 8
HD = 16
NRBF = 50
RBF_PAD = 56
CUT = 5.0
EPS = 1e-15
BETA = (2.0 / NRBF * (1.0 - np.exp(-CUT))) ** -2

NC = 2
NS = 16
NW = NC * NS
PW = NE // NW
PCH = 16
SUP = 2000
NPADN = 10240
NODES_W = NPADN // NW
NEB = 327680
AGGN = 10240

_MESH = dict(core_axis_name="c", subcore_axis_name="s", num_cores=NC,
             num_subcores=NS)
_SC_PARAMS = pltpu.CompilerParams(needs_layout_passes=False)


def _prep_body(ai_h, bi_h, px_h, py_h, pz_h, z_h, ae_h, ne_h,
               d2_h, x0_h, t_h,
               pxv, pyv, pzv, aiv, biv, d2v, zv, rows, sem):
    c = lax.axis_index("c")
    s = lax.axis_index("s")
    w = s * NC + c
    pltpu.sync_copy(px_h, pxv)
    pltpu.sync_copy(py_h, pyv)
    pltpu.sync_copy(pz_h, pzv)
    base = w * PW
    pltpu.sync_copy(ai_h.at[pl.ds(base, PW)], aiv)
    pltpu.sync_copy(bi_h.at[pl.ds(base, PW)], biv)

    def step(j, carry):
        o = j * 16
        ai = aiv[pl.ds(o, 16)]
        bi = biv[pl.ds(o, 16)]
        dx = plsc.load_gather(pxv, [ai]) - plsc.load_gather(pxv, [bi])
        dy = plsc.load_gather(pyv, [ai]) - plsc.load_gather(pyv, [bi])
        dz = plsc.load_gather(pzv, [ai]) - plsc.load_gather(pzv, [bi])
        d2v[pl.ds(o, 16)] = dx * dx + dy * dy + dz * dz
        return carry

    lax.fori_loop(0, PW // 16, step, 0)
    pltpu.sync_copy(d2v, d2_h.at[pl.ds(base, PW)])

    nb = w * NODES_W

    def nstep(i, carry):
        o = nb + i * 64
        pltpu.sync_copy(z_h.at[pl.ds(o, 64)], zv)
        pltpu.async_copy(ae_h.at[zv], rows, sem).wait()
        pltpu.sync_copy(rows, x0_h.at[pl.ds(o, 64)])
        pltpu.async_copy(ne_h.at[zv], rows, sem).wait()
        pltpu.sync_copy(rows, t_h.at[pl.ds(o, 64)])
        return carry

    lax.fori_loop(0, NODES_W // 64, nstep, 0)


def _sc_prep(ai, bi, px, py, pz, zpad, aemb, nemb):
    return pl.kernel(
        _prep_body,
        out_type=[
            jax.ShapeDtypeStruct((NE,), F32),
            jax.ShapeDtypeStruct((NPADN, FEAT), F32),
            jax.ShapeDtypeStruct((NPADN, FEAT), F32),
        ],
        mesh=plsc.VectorSubcoreMesh(**_MESH),
        compiler_params=_SC_PARAMS,
        scratch_types=[
            pltpu.VMEM((NN,), F32),
            pltpu.VMEM((NN,), F32),
            pltpu.VMEM((NN,), F32),
            pltpu.VMEM((PW,), I32),
            pltpu.VMEM((PW,), I32),
            pltpu.VMEM((PW,), F32),
            pltpu.VMEM((64,), I32),
            pltpu.VMEM((64, FEAT), F32),
            pltpu.SemaphoreType.DMA,
        ],
    )(ai, bi, px, py, pz, zpad, aemb, nemb)


def _zero_accum(s, zv, agg):
    zero16 = jnp.zeros((16,), F32)

    def zrow(i, carry):
        for h in range(NH):
            zv[i, pl.ds(h * 16, 16)] = zero16
        return carry

    lax.fori_loop(0, 32, zrow, 0)

    def zout(r, carry):
        pltpu.sync_copy(zv, agg.at[pl.ds(s * 640 + r * 32, 32)])
        return carry

    lax.fori_loop(0, 20, zout, 0)
    plsc.subcore_barrier()


def _write_accum(c, s, zv, agg, out_h):
    plsc.subcore_barrier()

    def orow(r, carry):
        rs = s * 640 + r * 32
        pltpu.sync_copy(agg.at[pl.ds(rs, 32)], zv)
        pltpu.sync_copy(zv, out_h.at[c, pl.ds(rs, 32)])
        return carry

    lax.fori_loop(0, 20, orow, 0)


def _emb_body(ai_h, bi_h, t_h, wed_h, out_h,
              iav, ibv, iaw, ibw, tA, tB, wv, mA, mB, zv, agg, sem):
    c = lax.axis_index("c")
    s = lax.axis_index("s")
    w = s * NC + c
    _zero_accum(s, zv, agg)

    def sup(o5, carry):
        base = w * PW + o5 * SUP
        pltpu.sync_copy(ai_h.at[pl.ds(base, SUP)], iav)
        pltpu.sync_copy(bi_h.at[pl.ds(base, SUP)], ibv)

        def chunk(i, carry2):
            off = base + i * PCH
            iaw[...] = iav[pl.ds(i * PCH, PCH)]
            ibw[...] = ibv[pl.ds(i * PCH, PCH)]
            ca = pltpu.async_copy(t_h.at[iaw], tA, sem)
            cb = pltpu.async_copy(t_h.at[ibw], tB, sem)
            pltpu.sync_copy(wed_h.at[pl.ds(off, PCH)], wv)
            ca.wait()
            cb.wait()

            @plsc.parallel_loop(0, PCH, unroll=4)
            def pair(p):
                for h in range(NH):
                    sl = pl.ds(h * 16, 16)
                    wvec = wv[p, sl]
                    mA[p, sl] = wvec * tB[p, sl]
                    mB[p, sl] = wvec * tA[p, sl]
            pltpu.sync_copy(mA, agg.at[iaw], add=True)
            pltpu.sync_copy(mB, agg.at[ibw], add=True)
            return carry2

        lax.fori_loop(0, SUP // PCH, chunk, 0)
        return carry

    lax.fori_loop(0, PW // SUP, sup, 0)
    _write_accum(c, s, zv, agg, out_h)


def _sc_emb_msg(ai, bi, t, wed):
    return pl.kernel(
        _emb_body,
        out_type=[jax.ShapeDtypeStruct((NC, AGGN, FEAT), F32)],
        mesh=plsc.VectorSubcoreMesh(**_MESH),
        compiler_params=_SC_PARAMS,
        scratch_types=[
            pltpu.VMEM((SUP,), I32),
            pltpu.VMEM((SUP,), I32),
            pltpu.VMEM((PCH,), I32),
            pltpu.VMEM((PCH,), I32),
            pltpu.VMEM((PCH, FEAT), F32),
            pltpu.VMEM((PCH, FEAT), F32),
            pltpu.VMEM((PCH, FEAT), F32),
            pltpu.VMEM((PCH, FEAT), F32),
            pltpu.VMEM((PCH, FEAT), F32),
            pltpu.VMEM((32, FEAT), F32),
            pltpu.VMEM_SHARED((AGGN, FEAT), F32),
            pltpu.SemaphoreType.DMA,
        ],
    )(ai, bi, t, wed)[0]


def _conv_body(ai_h, bi_h, qkv_h, dd_h, out_h,
               iav, ibv, iaw, ibw, gA, gB, ddv, mA, mB, zv, agg, sem):
    c = lax.axis_index("c")
    s = lax.axis_index("s")
    w = s * NC + c
    _zero_accum(s, zv, agg)

    def sup(o5, carry):
        base = w * PW + o5 * SUP
        pltpu.sync_copy(ai_h.at[pl.ds(base, SUP)], iav)
        pltpu.sync_copy(bi_h.at[pl.ds(base, SUP)], ibv)

        def chunk(i, carry2):
            off = base + i * PCH
            iaw[...] = iav[pl.ds(i * PCH, PCH)]
            ibw[...] = ibv[pl.ds(i * PCH, PCH)]
            ca = pltpu.async_copy(qkv_h.at[iaw], gA, sem)
            cb = pltpu.async_copy(qkv_h.at[ibw], gB, sem)
            pltpu.sync_copy(dd_h.at[pl.ds(off, PCH)], ddv)
            ca.wait()
            cb.wait()

            @plsc.parallel_loop(0, PCH, unroll=4)
            def pair(p):
                for h in range(NH):
                    o = h * 16
                    qa = gA[p, pl.ds(o, 16)]
                    ka = gA[p, pl.ds(FEAT + o, 16)]
                    va = gA[p, pl.ds(2 * FEAT + o, 16)]
                    qb = gB[p, pl.ds(o, 16)]
                    kb = gB[p, pl.ds(FEAT + o, 16)]
                    vb = gB[p, pl.ds(2 * FEAT + o, 16)]
                    dk = ddv[p, pl.ds(o, 16)]
                    dv = ddv[p, pl.ds(FEAT + o, 16)]
                    ra = jnp.sum(qa * kb * dk)
                    rb = jnp.sum(qb * ka * dk)
                    rav = jnp.full((16,), ra, F32)
                    rbv = jnp.full((16,), rb, F32)
                    sa = rav / (1.0 + jnp.exp(-rav))
                    sb = rbv / (1.0 + jnp.exp(-rbv))
                    mA[p, pl.ds(o, 16)] = vb * dv * sa
                    mB[p, pl.ds(o, 16)] = va * dv * sb
            pltpu.sync_copy(mA, agg.at[iaw], add=True)
            pltpu.sync_copy(mB, agg.at[ibw], add=True)
            return carry2

        lax.fori_loop(0, SUP // PCH, chunk, 0)
        return carry

    lax.fori_loop(0, PW // SUP, sup, 0)
    _write_accum(c, s, zv, agg, out_h)


def _sc_conv_msg(ai, bi, qkv, dd):
    return pl.kernel(
        _conv_body,
        out_type=[jax.ShapeDtypeStruct((NC, AGGN, FEAT), F32)],
        mesh=plsc.VectorSubcoreMesh(**_MESH),
        compiler_params=_SC_PARAMS,
        scratch_types=[
            pltpu.VMEM((SUP,), I32),
            pltpu.VMEM((SUP,), I32),
            pltpu.VMEM((PCH,), I32),
            pltpu.VMEM((PCH,), I32),
            pltpu.VMEM((PCH, 3 * FEAT), F32),
            pltpu.VMEM((PCH, 3 * FEAT), F32),
            pltpu.VMEM((PCH, 2 * FEAT), F32),
            pltpu.VMEM((PCH, FEAT), F32),
            pltpu.VMEM((PCH, FEAT), F32),
            pltpu.VMEM((32, FEAT), F32),
            pltpu.VMEM_SHARED((AGGN, FEAT), F32),
            pltpu.SemaphoreType.DMA,
        ],
    )(ai, bi, qkv, dd)[0]


def _silu(x):
    return x / (1.0 + jnp.exp(-x))


def _rbf_body(d2_r, mu_r, wW_r, wb_r, w1_r, b1_r, w2_r, b2_r, w3_r, b3_r,
              wed_o, d1_o, d2o_o, d3_o):
    eye = (lax.broadcasted_iota(I32, (128, 128), 0)
           == lax.broadcasted_iota(I32, (128, 128), 1)).astype(F32)
    cdims = (((0,), (0,)), ((), ()))
    for g in range(8):
        d2b = d2_r[g:g + 1, :]
        dist = jnp.sqrt(d2b + EPS)
        ed = jnp.exp(-dist)
        fc = jnp.where(dist < CUT,
                       0.5 * (jnp.cos(np.pi / CUT * dist) + 1.0), 0.0)
        erbT = fc * jnp.exp(-BETA * (ed - mu_r[...]) ** 2)
        fcc = lax.dot_general(eye, fc, (((1,), (1,)), ((), ())))
        rows = pl.ds(g * 128, 128)
        wed = _silu(lax.dot_general(erbT, wW_r[...], cdims) + wb_r[...])
        wed_o[rows, :] = wed * fcc
        for (wr, br, oo) in ((w1_r, b1_r, d1_o), (w2_r, b2_r, d2o_o),
                             (w3_r, b3_r, d3_o)):
            dd = _silu(lax.dot_general(erbT, wr[...], cdims) + br[...])
            oo[rows, 0:FEAT] = dd[:, 0:FEAT]
            oo[rows, FEAT:2 * FEAT] = dd[:, FEAT:2 * FEAT] * fcc


def _tc_rbf(d2p, mu_b, wW, wb, ws):
    n_blk = NEB // 1024
    full = lambda shp: pl.BlockSpec(shp, lambda b: (0, 0))
    outs = [jax.ShapeDtypeStruct((NEB, FEAT), F32)] + \
           [jax.ShapeDtypeStruct((NEB, 2 * FEAT), F32)] * 3
    return pl.pallas_call(
        _rbf_body,
        grid=(n_blk,),
        in_specs=[pl.BlockSpec((8, 128), lambda b: (b, 0)),
                  full((RBF_PAD, 128)), full((RBF_PAD, FEAT)), full((1, FEAT)),
                  full((RBF_PAD, 2 * FEAT)), full((1, 2 * FEAT)),
                  full((RBF_PAD, 2 * FEAT)), full((1, 2 * FEAT)),
                  full((RBF_PAD, 2 * FEAT)), full((1, 2 * FEAT))],
        out_specs=[pl.BlockSpec((1024, FEAT), lambda b: (b, 0)),
                   pl.BlockSpec((1024, 2 * FEAT), lambda b: (b, 0)),
                   pl.BlockSpec((1024, 2 * FEAT), lambda b: (b, 0)),
                   pl.BlockSpec((1024, 2 * FEAT), lambda b: (b, 0))],
        out_shape=outs,
    )(d2p, mu_b, wW, wb, ws[0][0], ws[0][1], ws[1][0], ws[1][1],
      ws[2][0], ws[2][1])


_MM = (((1,), (0,)), ((), ()))


def _embed_update_body(x0_r, m0_r, m1_r, cw_r, cb_r, qw_r, x_o, qkv_o):
    m = m0_r[...] + m1_r[...]
    x = (lax.dot_general(x0_r[...], cw_r[0:FEAT, :], _MM)
         + lax.dot_general(m, cw_r[FEAT:2 * FEAT, :], _MM) + cb_r[...])
    x_o[...] = x
    qkv_o[...] = lax.dot_general(x, qw_r[...], _MM)


def _tc_embed_update(x0, m0, m1, cw, cb, qw):
    return pl.pallas_call(
        _embed_update_body,
        out_shape=[jax.ShapeDtypeStruct((NN, FEAT), F32),
                   jax.ShapeDtypeStruct((NN, 3 * FEAT), F32)],
    )(x0, m0, m1, cw, cb, qw)


def _layer_update_body(x_r, a0_r, a1_r, ow_r, ob_r, qw_r, x_o, qkv_o):
    agg = a0_r[...] + a1_r[...]
    x = x_r[...] + lax.dot_general(agg, ow_r[...], _MM) + ob_r[...]
    x_o[...] = x
    qkv_o[...] = lax.dot_general(x, qw_r[...], _MM)


def _tc_layer_update(x, a0, a1, ow, ob, qw):
    return pl.pallas_call(
        _layer_update_body,
        out_shape=[jax.ShapeDtypeStruct((NN, FEAT), F32),
                   jax.ShapeDtypeStruct((NN, 3 * FEAT), F32)],
    )(x, a0, a1, ow, ob, qw)


def _readout_body(x_r, a0_r, a1_r, ow_r, ob_r, rw1_r, rb1_r, rw2_r, rb2_r,
                  mol_r, out_o):
    agg = a0_r[...] + a1_r[...]
    x = x_r[...] + lax.dot_general(agg, ow_r[...], _MM) + ob_r[...]
    h = _silu(lax.dot_general(x, rw1_r[...], _MM) + rb1_r[...])
    ae = jnp.sum(h * rw2_r[...], axis=1, keepdims=True) + rb2_r[...]
    iota = lax.broadcasted_iota(I32, (NN, 128), 1)
    mask = (mol_r[...] == iota).astype(F32)
    out_o[...] = jnp.sum(mask * ae, axis=0, keepdims=True)


def _tc_readout(x, a0, a1, ow, ob, rw1, rb1, rw2t, rb2, molb):
    return pl.pallas_call(
        _readout_body,
        out_shape=jax.ShapeDtypeStruct((1, 128), F32),
    )(x, a0, a1, ow, ob, rw1, rb1, rw2t, rb2, molb)


def kernel(nxyz, params, nbr_list, mol_id):
    pos = nxyz[:, 1:]
    z = jnp.clip(nxyz[:, 0], 0, 99).astype(I32)
    zpad = jnp.concatenate([z, jnp.zeros((NPADN - NN,), I32)])
    px, py, pz = pos[:, 0], pos[:, 1], pos[:, 2]
    ai = nbr_list[:, 0].astype(I32)
    bi = nbr_list[:, 1].astype(I32)

    d2, x0p, t = _sc_prep(ai, bi, px, py, pz, zpad,
                          params['atom_emb'], params['nbr_emb'])
    x0 = x0p[:NN]

    d2p = jnp.concatenate([d2, jnp.full((NEB - NE,), 1e6, F32)])
    d2p = d2p.reshape(NEB // 128, 128)

    mu = np.zeros((RBF_PAD,), np.float32)
    mu[:NRBF] = np.linspace(np.exp(-CUT), 1.0, NRBF)
    mu_b = jnp.broadcast_to(jnp.asarray(mu)[:, None], (RBF_PAD, 128))

    pad_w = lambda w: jnp.pad(w, ((0, RBF_PAD - NRBF), (0, 0)))
    wW = pad_w(params['rbf_W'])
    wb = params['rbf_b'][None, :]
    ws = [(pad_w(jnp.concatenate([l['dk_W'], l['dv_W']], axis=1)),
           jnp.concatenate([l['dk_b'], l['dv_b']])[None, :])
          for l in params['convs']]

    wed, dd1, dd2, dd3 = _tc_rbf(d2p, mu_b, wW, wb, ws)

    msg0 = _sc_emb_msg(ai, bi, t, wed[:NE])[:, :NN]
    qkvw = [jnp.concatenate([l['q_W'], l['k_W'], l['v_W']], axis=1)
            for l in params['convs']]
    x, qkv = _tc_embed_update(x0, msg0[0], msg0[1],
                              params['comb_W'], params['comb_b'][None, :],
                              qkvw[0])

    dds = [dd1[:NE], dd2[:NE], dd3[:NE]]
    for l in range(2):
        aggp = _sc_conv_msg(ai, bi, qkv, dds[l])[:, :NN]
        x, qkv = _tc_layer_update(x, aggp[0], aggp[1],
                                  params['convs'][l]['o_W'],
                                  params['convs'][l]['o_b'][None, :],
                                  qkvw[l + 1])
    aggp = _sc_conv_msg(ai, bi, qkv, dds[2])[:, :NN]
    molb = jnp.broadcast_to(mol_id.astype(I32)[:, None], (NN, 128))
    out = _tc_readout(x, aggp[0], aggp[1],
                      params['convs'][2]['o_W'],
                      params['convs'][2]['o_b'][None, :],
                      params['r_W1'], params['r_b1'][None, :],
                      params['r_W2'][:, 0][None, :],
                      params['r_b2'][None, :], molb)
    return out[0, :100]


# speedup vs baseline: 1.8048x; 1.8048x over previous
import functools

import jax
import jax.numpy as jnp
import numpy as np
from jax import lax
from jax.experimental import pallas as pl
from jax.experimental.pallas import tpu as pltpu
from jax.experimental.pallas import tpu_sc as plsc

F32 = jnp.float32
I32 = jnp.int32

NN = 10000
NE = 320000
FEAT = 128
NH = 8
HD = 16
NRBF = 50
RBF_PAD = 56
CUT = 5.0
EPS = 1e-15
BETA = (2.0 / NRBF * (1.0 - np.exp(-CUT))) ** -2

NC = 2
NS = 16
NW = NC * NS
PW = NE // NW
PCH = 16
SUP = 2000
NPADN = 10240
NODES_W = NPADN // NW
NEB = 327680
AGGN = 10240

_MESH = dict(core_axis_name="c", subcore_axis_name="s", num_cores=NC,
             num_subcores=NS)
_SC_PARAMS = pltpu.CompilerParams(needs_layout_passes=False)


def _prep_body(ai_h, bi_h, px_h, py_h, pz_h, z_h, ae_h, ne_h,
               d2_h, x0_h, t_h,
               pxv, pyv, pzv, aiv, biv, d2v, zv, rows, sem):
    c = lax.axis_index("c")
    s = lax.axis_index("s")
    w = s * NC + c
    pltpu.sync_copy(px_h, pxv)
    pltpu.sync_copy(py_h, pyv)
    pltpu.sync_copy(pz_h, pzv)
    base = w * PW
    pltpu.sync_copy(ai_h.at[pl.ds(base, PW)], aiv)
    pltpu.sync_copy(bi_h.at[pl.ds(base, PW)], biv)

    def step(j, carry):
        o = j * 16
        ai = aiv[pl.ds(o, 16)]
        bi = biv[pl.ds(o, 16)]
        dx = plsc.load_gather(pxv, [ai]) - plsc.load_gather(pxv, [bi])
        dy = plsc.load_gather(pyv, [ai]) - plsc.load_gather(pyv, [bi])
        dz = plsc.load_gather(pzv, [ai]) - plsc.load_gather(pzv, [bi])
        d2v[pl.ds(o, 16)] = dx * dx + dy * dy + dz * dz
        return carry

    lax.fori_loop(0, PW // 16, step, 0)
    pltpu.sync_copy(d2v, d2_h.at[pl.ds(base, PW)])

    nb = w * NODES_W

    def nstep(i, carry):
        o = nb + i * 64
        pltpu.sync_copy(z_h.at[pl.ds(o, 64)], zv)
        pltpu.async_copy(ae_h.at[zv], rows, sem).wait()
        pltpu.sync_copy(rows, x0_h.at[pl.ds(o, 64)])
        pltpu.async_copy(ne_h.at[zv], rows, sem).wait()
        pltpu.sync_copy(rows, t_h.at[pl.ds(o, 64)])
        return carry

    lax.fori_loop(0, NODES_W // 64, nstep, 0)


def _sc_prep(ai, bi, px, py, pz, zpad, aemb, nemb):
    return pl.kernel(
        _prep_body,
        out_type=[
            jax.ShapeDtypeStruct((NE,), F32),
            jax.ShapeDtypeStruct((NPADN, FEAT), F32),
            jax.ShapeDtypeStruct((NPADN, FEAT), F32),
        ],
        mesh=plsc.VectorSubcoreMesh(**_MESH),
        compiler_params=_SC_PARAMS,
        scratch_types=[
            pltpu.VMEM((NN,), F32),
            pltpu.VMEM((NN,), F32),
            pltpu.VMEM((NN,), F32),
            pltpu.VMEM((PW,), I32),
            pltpu.VMEM((PW,), I32),
            pltpu.VMEM((PW,), F32),
            pltpu.VMEM((64,), I32),
            pltpu.VMEM((64, FEAT), F32),
            pltpu.SemaphoreType.DMA,
        ],
    )(ai, bi, px, py, pz, zpad, aemb, nemb)


def _zero_accum(s, zv, agg):
    zero16 = jnp.zeros((16,), F32)

    def zrow(i, carry):
        for h in range(NH):
            zv[i, pl.ds(h * 16, 16)] = zero16
        return carry

    lax.fori_loop(0, PCH, zrow, 0)

    def zout(r, carry):
        pltpu.sync_copy(zv, agg.at[pl.ds(s * 640 + r * 16, 16)])
        return carry

    lax.fori_loop(0, 40, zout, 0)
    plsc.subcore_barrier()


def _write_accum(c, s, zv, agg, out_h):
    plsc.subcore_barrier()

    def orow(r, carry):
        rs = s * 640 + r * 16
        pltpu.sync_copy(agg.at[pl.ds(rs, 16)], zv)
        pltpu.sync_copy(zv, out_h.at[c, pl.ds(rs, 16)])
        return carry

    lax.fori_loop(0, 40, orow, 0)


def _emb_body(ai_h, bi_h, t_h, wed_h, out_h,
              iav, ibv, iawA, ibwA, iawB, ibwB,
              tAA, tBA, wvA, tAB, tBB, wvB, mA, mB, agg, semA, semB):
    c = lax.axis_index("c")
    s = lax.axis_index("s")
    w = s * NC + c
    _zero_accum(s, mA, agg)

    def sup(o5, carry):
        base = w * PW + o5 * SUP
        pltpu.sync_copy(ai_h.at[pl.ds(base, SUP)], iav)
        pltpu.sync_copy(bi_h.at[pl.ds(base, SUP)], ibv)

        def issue(ci, iaw, ibw, tA, tB, wv, sem):
            iaw[...] = iav[pl.ds(ci * PCH, PCH)]
            ibw[...] = ibv[pl.ds(ci * PCH, PCH)]
            pltpu.make_async_copy(t_h.at[iaw], tA, sem).start()
            pltpu.make_async_copy(t_h.at[ibw], tB, sem).start()
            pltpu.make_async_copy(wed_h.at[pl.ds(base + ci * PCH, PCH)],
                                  wv, sem).start()

        def finish(iaw, ibw, tA, tB, wv, sem):
            pltpu.make_async_copy(t_h.at[iaw], tA, sem).wait()
            pltpu.make_async_copy(t_h.at[ibw], tB, sem).wait()
            pltpu.make_async_copy(wed_h.at[pl.ds(base, PCH)], wv, sem).wait()

            @plsc.parallel_loop(0, PCH, unroll=2)
            def pair(p):
                for h in range(NH):
                    sl = pl.ds(h * 16, 16)
                    wvec = wv[p, sl]
                    mA[p, sl] = wvec * tB[p, sl]
                    mB[p, sl] = wvec * tA[p, sl]
            pltpu.sync_copy(mA, agg.at[iaw], add=True)
            pltpu.sync_copy(mB, agg.at[ibw], add=True)

        issue(0, iawA, ibwA, tAA, tBA, wvA, semA)

        def dchunk(j, carry2):
            issue(2 * j + 1, iawB, ibwB, tAB, tBB, wvB, semB)
            finish(iawA, ibwA, tAA, tBA, wvA, semA)
            issue(2 * j + 2, iawA, ibwA, tAA, tBA, wvA, semA)
            finish(iawB, ibwB, tAB, tBB, wvB, semB)
            return carry2

        lax.fori_loop(0, (SUP // PCH) // 2, dchunk, 0)
        finish(iawA, ibwA, tAA, tBA, wvA, semA)
        return carry

    lax.fori_loop(0, PW // SUP, sup, 0)
    _write_accum(c, s, mA, agg, out_h)


def _sc_emb_msg(ai, bi, t, wed):
    return pl.kernel(
        _emb_body,
        out_type=[jax.ShapeDtypeStruct((NC, AGGN, FEAT), F32)],
        mesh=plsc.VectorSubcoreMesh(**_MESH),
        compiler_params=_SC_PARAMS,
        scratch_types=[
            pltpu.VMEM((SUP,), I32),
            pltpu.VMEM((SUP,), I32),
            pltpu.VMEM((PCH,), I32),
            pltpu.VMEM((PCH,), I32),
            pltpu.VMEM((PCH,), I32),
            pltpu.VMEM((PCH,), I32),
            pltpu.VMEM((PCH, FEAT), F32),
            pltpu.VMEM((PCH, FEAT), F32),
            pltpu.VMEM((PCH, FEAT), F32),
            pltpu.VMEM((PCH, FEAT), F32),
            pltpu.VMEM((PCH, FEAT), F32),
            pltpu.VMEM((PCH, FEAT), F32),
            pltpu.VMEM((PCH, FEAT), F32),
            pltpu.VMEM((PCH, FEAT), F32),
            pltpu.VMEM_SHARED((AGGN, FEAT), F32),
            pltpu.SemaphoreType.DMA,
            pltpu.SemaphoreType.DMA,
        ],
    )(ai, bi, t, wed)[0]


def _conv_body(ai_h, bi_h, qkv_h, dd_h, out_h,
               iav, ibv, iawA, ibwA, iawB, ibwB,
               gAA, gBA, ddA, gAB, gBB, ddB, mA, mB, agg, semA, semB):
    c = lax.axis_index("c")
    s = lax.axis_index("s")
    w = s * NC + c
    _zero_accum(s, mA, agg)

    def sup(o5, carry):
        base = w * PW + o5 * SUP
        pltpu.sync_copy(ai_h.at[pl.ds(base, SUP)], iav)
        pltpu.sync_copy(bi_h.at[pl.ds(base, SUP)], ibv)

        def issue(ci, iaw, ibw, gA, gB, ddv, sem):
            iaw[...] = iav[pl.ds(ci * PCH, PCH)]
            ibw[...] = ibv[pl.ds(ci * PCH, PCH)]
            pltpu.make_async_copy(qkv_h.at[iaw], gA, sem).start()
            pltpu.make_async_copy(qkv_h.at[ibw], gB, sem).start()
            pltpu.make_async_copy(dd_h.at[pl.ds(base + ci * PCH, PCH)],
                                  ddv, sem).start()

        def finish(iaw, ibw, gA, gB, ddv, sem):
            pltpu.make_async_copy(qkv_h.at[iaw], gA, sem).wait()
            pltpu.make_async_copy(qkv_h.at[ibw], gB, sem).wait()
            pltpu.make_async_copy(dd_h.at[pl.ds(base, PCH)], ddv, sem).wait()

            @plsc.parallel_loop(0, PCH, unroll=2)
            def pair(p):
                for h in range(NH):
                    o = h * 16
                    qa = gA[p, pl.ds(o, 16)]
                    ka = gA[p, pl.ds(FEAT + o, 16)]
                    va = gA[p, pl.ds(2 * FEAT + o, 16)]
                    qb = gB[p, pl.ds(o, 16)]
                    kb = gB[p, pl.ds(FEAT + o, 16)]
                    vb = gB[p, pl.ds(2 * FEAT + o, 16)]
                    dk = ddv[p, pl.ds(o, 16)]
                    dv = ddv[p, pl.ds(FEAT + o, 16)]
                    ra = jnp.sum(qa * kb * dk)
                    rb = jnp.sum(qb * ka * dk)
                    rav = jnp.full((16,), ra, F32)
                    rbv = jnp.full((16,), rb, F32)
                    sa = rav / (1.0 + jnp.exp(-rav))
                    sb = rbv / (1.0 + jnp.exp(-rbv))
                    mA[p, pl.ds(o, 16)] = vb * dv * sa
                    mB[p, pl.ds(o, 16)] = va * dv * sb
            pltpu.sync_copy(mA, agg.at[iaw], add=True)
            pltpu.sync_copy(mB, agg.at[ibw], add=True)

        issue(0, iawA, ibwA, gAA, gBA, ddA, semA)

        def dchunk(j, carry2):
            issue(2 * j + 1, iawB, ibwB, gAB, gBB, ddB, semB)
            finish(iawA, ibwA, gAA, gBA, ddA, semA)
            issue(2 * j + 2, iawA, ibwA, gAA, gBA, ddA, semA)
            finish(iawB, ibwB, gAB, gBB, ddB, semB)
            return carry2

        lax.fori_loop(0, (SUP // PCH) // 2, dchunk, 0)
        finish(iawA, ibwA, gAA, gBA, ddA, semA)
        return carry

    lax.fori_loop(0, PW // SUP, sup, 0)
    _write_accum(c, s, mA, agg, out_h)


def _sc_conv_msg(ai, bi, qkv, dd):
    return pl.kernel(
        _conv_body,
        out_type=[jax.ShapeDtypeStruct((NC, AGGN, FEAT), F32)],
        mesh=plsc.VectorSubcoreMesh(**_MESH),
        compiler_params=_SC_PARAMS,
        scratch_types=[
            pltpu.VMEM((SUP,), I32),
            pltpu.VMEM((SUP,), I32),
            pltpu.VMEM((PCH,), I32),
            pltpu.VMEM((PCH,), I32),
            pltpu.VMEM((PCH,), I32),
            pltpu.VMEM((PCH,), I32),
            pltpu.VMEM((PCH, 3 * FEAT), F32),
            pltpu.VMEM((PCH, 3 * FEAT), F32),
            pltpu.VMEM((PCH, 2 * FEAT), F32),
            pltpu.VMEM((PCH, 3 * FEAT), F32),
            pltpu.VMEM((PCH, 3 * FEAT), F32),
            pltpu.VMEM((PCH, 2 * FEAT), F32),
            pltpu.VMEM((PCH, FEAT), F32),
            pltpu.VMEM((PCH, FEAT), F32),
            pltpu.VMEM_SHARED((AGGN, FEAT), F32),
            pltpu.SemaphoreType.DMA,
            pltpu.SemaphoreType.DMA,
        ],
    )(ai, bi, qkv, dd)[0]


def _silu(x):
    return x / (1.0 + jnp.exp(-x))


def _rbf_body(d2_r, mu_r, wW_r, wb_r, w1_r, b1_r, w2_r, b2_r, w3_r, b3_r,
              wed_o, d1_o, d2o_o, d3_o):
    eye = (lax.broadcasted_iota(I32, (128, 128), 0)
           == lax.broadcasted_iota(I32, (128, 128), 1)).astype(F32)
    cdims = (((0,), (0,)), ((), ()))
    for g in range(8):
        d2b = d2_r[g:g + 1, :]
        dist = jnp.sqrt(d2b + EPS)
        ed = jnp.exp(-dist)
        fc = jnp.where(dist < CUT,
                       0.5 * (jnp.cos(np.pi / CUT * dist) + 1.0), 0.0)
        erbT = fc * jnp.exp(-BETA * (ed - mu_r[...]) ** 2)
        fcc = lax.dot_general(eye, fc, (((1,), (1,)), ((), ())))
        rows = pl.ds(g * 128, 128)
        wed = _silu(lax.dot_general(erbT, wW_r[...], cdims) + wb_r[...])
        wed_o[rows, :] = wed * fcc
        for (wr, br, oo) in ((w1_r, b1_r, d1_o), (w2_r, b2_r, d2o_o),
                             (w3_r, b3_r, d3_o)):
            dd = _silu(lax.dot_general(erbT, wr[...], cdims) + br[...])
            oo[rows, 0:FEAT] = dd[:, 0:FEAT]
            oo[rows, FEAT:2 * FEAT] = dd[:, FEAT:2 * FEAT] * fcc


def _tc_rbf(d2p, mu_b, wW, wb, ws):
    n_blk = NEB // 1024
    full = lambda shp: pl.BlockSpec(shp, lambda b: (0, 0))
    outs = [jax.ShapeDtypeStruct((NEB, FEAT), F32)] + \
           [jax.ShapeDtypeStruct((NEB, 2 * FEAT), F32)] * 3
    return pl.pallas_call(
        _rbf_body,
        grid=(n_blk,),
        in_specs=[pl.BlockSpec((8, 128), lambda b: (b, 0)),
                  full((RBF_PAD, 128)), full((RBF_PAD, FEAT)), full((1, FEAT)),
                  full((RBF_PAD, 2 * FEAT)), full((1, 2 * FEAT)),
                  full((RBF_PAD, 2 * FEAT)), full((1, 2 * FEAT)),
                  full((RBF_PAD, 2 * FEAT)), full((1, 2 * FEAT))],
        out_specs=[pl.BlockSpec((1024, FEAT), lambda b: (b, 0)),
                   pl.BlockSpec((1024, 2 * FEAT), lambda b: (b, 0)),
                   pl.BlockSpec((1024, 2 * FEAT), lambda b: (b, 0)),
                   pl.BlockSpec((1024, 2 * FEAT), lambda b: (b, 0))],
        out_shape=outs,
    )(d2p, mu_b, wW, wb, ws[0][0], ws[0][1], ws[1][0], ws[1][1],
      ws[2][0], ws[2][1])


_MM = (((1,), (0,)), ((), ()))


def _embed_update_body(x0_r, m0_r, m1_r, cw_r, cb_r, qw_r, x_o, qkv_o):
    m = m0_r[...] + m1_r[...]
    x = (lax.dot_general(x0_r[...], cw_r[0:FEAT, :], _MM)
         + lax.dot_general(m, cw_r[FEAT:2 * FEAT, :], _MM) + cb_r[...])
    x_o[...] = x
    qkv_o[...] = lax.dot_general(x, qw_r[...], _MM)


def _tc_embed_update(x0, m0, m1, cw, cb, qw):
    return pl.pallas_call(
        _embed_update_body,
        out_shape=[jax.ShapeDtypeStruct((NN, FEAT), F32),
                   jax.ShapeDtypeStruct((NN, 3 * FEAT), F32)],
    )(x0, m0, m1, cw, cb, qw)


def _layer_update_body(x_r, a0_r, a1_r, ow_r, ob_r, qw_r, x_o, qkv_o):
    agg = a0_r[...] + a1_r[...]
    x = x_r[...] + lax.dot_general(agg, ow_r[...], _MM) + ob_r[...]
    x_o[...] = x
    qkv_o[...] = lax.dot_general(x, qw_r[...], _MM)


def _tc_layer_update(x, a0, a1, ow, ob, qw):
    return pl.pallas_call(
        _layer_update_body,
        out_shape=[jax.ShapeDtypeStruct((NN, FEAT), F32),
                   jax.ShapeDtypeStruct((NN, 3 * FEAT), F32)],
    )(x, a0, a1, ow, ob, qw)


def _readout_body(x_r, a0_r, a1_r, ow_r, ob_r, rw1_r, rb1_r, rw2_r, rb2_r,
                  mol_r, out_o):
    agg = a0_r[...] + a1_r[...]
    x = x_r[...] + lax.dot_general(agg, ow_r[...], _MM) + ob_r[...]
    h = _silu(lax.dot_general(x, rw1_r[...], _MM) + rb1_r[...])
    ae = jnp.sum(h * rw2_r[...], axis=1, keepdims=True) + rb2_r[...]
    iota = lax.broadcasted_iota(I32, (NN, 128), 1)
    mask = (mol_r[...] == iota).astype(F32)
    out_o[...] = jnp.sum(mask * ae, axis=0, keepdims=True)


def _tc_readout(x, a0, a1, ow, ob, rw1, rb1, rw2t, rb2, molb):
    return pl.pallas_call(
        _readout_body,
        out_shape=jax.ShapeDtypeStruct((1, 128), F32),
    )(x, a0, a1, ow, ob, rw1, rb1, rw2t, rb2, molb)


def kernel(nxyz, params, nbr_list, mol_id):
    pos = nxyz[:, 1:]
    z = jnp.clip(nxyz[:, 0], 0, 99).astype(I32)
    zpad = jnp.concatenate([z, jnp.zeros((NPADN - NN,), I32)])
    px, py, pz = pos[:, 0], pos[:, 1], pos[:, 2]
    ai = nbr_list[:, 0].astype(I32)
    bi = nbr_list[:, 1].astype(I32)

    d2, x0p, t = _sc_prep(ai, bi, px, py, pz, zpad,
                          params['atom_emb'], params['nbr_emb'])
    x0 = x0p[:NN]

    d2p = jnp.concatenate([d2, jnp.full((NEB - NE,), 1e6, F32)])
    d2p = d2p.reshape(NEB // 128, 128)

    mu = np.zeros((RBF_PAD,), np.float32)
    mu[:NRBF] = np.linspace(np.exp(-CUT), 1.0, NRBF)
    mu_b = jnp.broadcast_to(jnp.asarray(mu)[:, None], (RBF_PAD, 128))

    pad_w = lambda w: jnp.pad(w, ((0, RBF_PAD - NRBF), (0, 0)))
    wW = pad_w(params['rbf_W'])
    wb = params['rbf_b'][None, :]
    ws = [(pad_w(jnp.concatenate([l['dk_W'], l['dv_W']], axis=1)),
           jnp.concatenate([l['dk_b'], l['dv_b']])[None, :])
          for l in params['convs']]

    wed, dd1, dd2, dd3 = _tc_rbf(d2p, mu_b, wW, wb, ws)

    msg0 = _sc_emb_msg(ai, bi, t, wed[:NE])[:, :NN]
    qkvw = [jnp.concatenate([l['q_W'], l['k_W'], l['v_W']], axis=1)
            for l in params['convs']]
    x, qkv = _tc_embed_update(x0, msg0[0], msg0[1],
                              params['comb_W'], params['comb_b'][None, :],
                              qkvw[0])

    dds = [dd1[:NE], dd2[:NE], dd3[:NE]]
    for l in range(2):
        aggp = _sc_conv_msg(ai, bi, qkv, dds[l])[:, :NN]
        x, qkv = _tc_layer_update(x, aggp[0], aggp[1],
                                  params['convs'][l]['o_W'],
                                  params['convs'][l]['o_b'][None, :],
                                  qkvw[l + 1])
    aggp = _sc_conv_msg(ai, bi, qkv, dds[2])[:, :NN]
    molb = jnp.broadcast_to(mol_id.astype(I32)[:, None], (NN, 128))
    out = _tc_readout(x, aggp[0], aggp[1],
                      params['convs'][2]['o_W'],
                      params['convs'][2]['o_b'][None, :],
                      params['r_W1'], params['r_b1'][None, :],
                      params['r_W2'][:, 0][None, :],
                      params['r_b2'][None, :], molb)
    return out[0, :100]
